# baseline (speedup 1.0000x reference)
import functools
import math

import jax
import jax.numpy as jnp
from jax import lax
from jax.experimental import pallas as pl
from jax.experimental.pallas import tpu as pltpu
from jax.experimental.pallas import tpu_sc as plsc

D_MODEL = 2048
_SCALE = math.sqrt(D_MODEL)
_LANES = 16
_C = 16
_NBUF = 3


@functools.lru_cache(maxsize=None)
def _make_gather(B, V):
    info = plsc.get_sparse_core_info()
    nc, ns = info.num_cores, info.num_subcores
    nw = nc * ns
    b_per_w = B // nw
    n_chunks = b_per_w // _C

    mesh = plsc.VectorSubcoreMesh(core_axis_name="c", subcore_axis_name="s")

    row_buf = pltpu.VMEM((_C, D_MODEL), jnp.float32)

    @functools.partial(
        pl.kernel,
        mesh=mesh,
        out_type=jax.ShapeDtypeStruct((B, D_MODEL), jnp.float32),
        scratch_types=[
            pltpu.VMEM((n_chunks, _C), jnp.int32),
            row_buf, row_buf, row_buf,
            pltpu.SemaphoreType.DMA, pltpu.SemaphoreType.DMA,
            pltpu.SemaphoreType.DMA, pltpu.SemaphoreType.DMA,
            pltpu.SemaphoreType.DMA, pltpu.SemaphoreType.DMA,
        ],
    )
    def k(x_hbm, lut_hbm, out_hbm, idx_v, r0, r1, r2,
          g0, g1, g2, s0, s1, s2):
        rows = (r0, r1, r2)
        gsem = (g0, g1, g2)
        ssem = (s0, s1, s2)
        wid = lax.axis_index("s") * nc + lax.axis_index("c")
        base = wid * b_per_w
        pltpu.sync_copy(x_hbm.at[wid], idx_v)

        def scale(buf):
            for r in range(_C):
                @plsc.parallel_loop(0, D_MODEL, step=_LANES, unroll=8)
                def _(i):
                    buf[r, pl.ds(i, _LANES)] = buf[r, pl.ds(i, _LANES)] * _SCALE

        def issue_gather(h, b):
            pltpu.async_copy(lut_hbm.at[idx_v.at[h]], rows[b], gsem[b])

        def step(g, b, first_step, issue_next):
            pltpu.make_async_copy(
                lut_hbm.at[pl.ds(0, _C)], rows[b], gsem[b]
            ).wait()
            scale(rows[b])
            pltpu.async_copy(
                rows[b], out_hbm.at[pl.ds(base + g * _C, _C)], ssem[b]
            )
            if issue_next:
                bh = (b + _NBUF - 1) % _NBUF
                if not first_step:
                    pltpu.make_async_copy(
                        rows[bh], out_hbm.at[pl.ds(0, _C)], ssem[bh]
                    ).wait()
                issue_gather(g + _NBUF - 1, bh)

        for h in range(_NBUF - 1):
            issue_gather(h, h % _NBUF)

        for tt in range(_NBUF):
            step(tt, tt, tt == 0, True)

        n_tail = (n_chunks - _NBUF) % _NBUF + _NBUF
        outer_hi = (n_chunks - n_tail) // _NBUF

        def outer_body(o, carry):
            for tt in range(_NBUF):
                step(o * _NBUF + tt, tt, False, True)
            return carry

        lax.fori_loop(1, outer_hi, outer_body, 0)

        for j in range(n_tail):
            g = n_chunks - n_tail + j
            step(g, g % _NBUF, False, g + _NBUF - 1 < n_chunks)

        for b in range(_NBUF):
            pltpu.make_async_copy(
                rows[b], out_hbm.at[pl.ds(0, _C)], ssem[b]
            ).wait()

    return k


def kernel(x, lut):
    b0, b1 = x.shape
    info = plsc.get_sparse_core_info()
    nw = info.num_cores * info.num_subcores
    B = b0 * b1
    idx3 = x.reshape(nw, (B // nw) // _C, _C).astype(jnp.int32)
    out = _make_gather(B, lut.shape[0])(idx3, lut)
    return out.reshape(b0, b1, D_MODEL)


# speedup vs baseline: 1.0067x; 1.0067x over previous
import functools
import math

import jax
import jax.numpy as jnp
from jax import lax
from jax.experimental import pallas as pl
from jax.experimental.pallas import tpu as pltpu
from jax.experimental.pallas import tpu_sc as plsc

D_MODEL = 2048
_SCALE = math.sqrt(D_MODEL)
_LANES = 16
_C = 8
_NBUF = 5


@functools.lru_cache(maxsize=None)
def _make_gather(B, V):
    info = plsc.get_sparse_core_info()
    nc, ns = info.num_cores, info.num_subcores
    nw = nc * ns
    b_per_w = B // nw
    n_chunks = b_per_w // _C

    mesh = plsc.VectorSubcoreMesh(core_axis_name="c", subcore_axis_name="s")

    row_buf = pltpu.VMEM((_C, D_MODEL), jnp.float32)

    @functools.partial(
        pl.kernel,
        mesh=mesh,
        out_type=jax.ShapeDtypeStruct((B, D_MODEL), jnp.float32),
        scratch_types=(
            [pltpu.VMEM((n_chunks, _C), jnp.int32)]
            + [row_buf] * _NBUF
            + [pltpu.SemaphoreType.DMA] * (2 * _NBUF)
        ),
    )
    def k(x_hbm, lut_hbm, out_hbm, idx_v, *bufs):
        rows = bufs[:_NBUF]
        gsem = bufs[_NBUF:2 * _NBUF]
        ssem = bufs[2 * _NBUF:]
        wid = lax.axis_index("s") * nc + lax.axis_index("c")
        base = wid * b_per_w
        pltpu.sync_copy(x_hbm.at[wid], idx_v)

        def scale(buf):
            for r in range(_C):
                @plsc.parallel_loop(0, D_MODEL, step=_LANES, unroll=8)
                def _(i):
                    buf[r, pl.ds(i, _LANES)] = buf[r, pl.ds(i, _LANES)] * _SCALE

        def issue_gather(h, b):
            pltpu.async_copy(lut_hbm.at[idx_v.at[h]], rows[b], gsem[b])

        def step(g, b, first_step, issue_next):
            pltpu.make_async_copy(
                lut_hbm.at[pl.ds(0, _C)], rows[b], gsem[b]
            ).wait()
            scale(rows[b])
            pltpu.async_copy(
                rows[b], out_hbm.at[pl.ds(base + g * _C, _C)], ssem[b]
            )
            if issue_next:
                bh = (b + _NBUF - 1) % _NBUF
                if not first_step:
                    pltpu.make_async_copy(
                        rows[bh], out_hbm.at[pl.ds(0, _C)], ssem[bh]
                    ).wait()
                issue_gather(g + _NBUF - 1, bh)

        for h in range(_NBUF - 1):
            issue_gather(h, h % _NBUF)

        for tt in range(_NBUF):
            step(tt, tt, tt == 0, True)

        n_tail = (n_chunks - _NBUF) % _NBUF + _NBUF
        outer_hi = (n_chunks - n_tail) // _NBUF

        def outer_body(o, carry):
            for tt in range(_NBUF):
                step(o * _NBUF + tt, tt, False, True)
            return carry

        lax.fori_loop(1, outer_hi, outer_body, 0)

        for j in range(n_tail):
            g = n_chunks - n_tail + j
            step(g, g % _NBUF, False, g + _NBUF - 1 < n_chunks)

        for b in range(_NBUF):
            pltpu.make_async_copy(
                rows[b], out_hbm.at[pl.ds(0, _C)], ssem[b]
            ).wait()

    return k


def kernel(x, lut):
    b0, b1 = x.shape
    info = plsc.get_sparse_core_info()
    nw = info.num_cores * info.num_subcores
    B = b0 * b1
    idx3 = x.reshape(nw, (B // nw) // _C, _C).astype(jnp.int32)
    out = _make_gather(B, lut.shape[0])(idx3, lut)
    return out.reshape(b0, b1, D_MODEL)


# speedup vs baseline: 1.0227x; 1.0159x over previous
import functools
import math

import jax
import jax.numpy as jnp
from jax import lax
from jax.experimental import pallas as pl
from jax.experimental.pallas import tpu as pltpu
from jax.experimental.pallas import tpu_sc as plsc

D_MODEL = 2048
_SCALE = math.sqrt(D_MODEL)
_LANES = 16
_C = 8
_NBUF = 4
_INNER = 4


@functools.lru_cache(maxsize=None)
def _make_gather(B, V):
    info = plsc.get_sparse_core_info()
    nc, ns = info.num_cores, info.num_subcores
    nw = nc * ns
    b_per_w = B // nw
    n_chunks = b_per_w // _C
    outer = n_chunks // _INNER

    mesh = plsc.VectorSubcoreMesh(core_axis_name="c", subcore_axis_name="s")

    row_buf = pltpu.VMEM((_C, D_MODEL), jnp.float32)

    @functools.partial(
        pl.kernel,
        mesh=mesh,
        out_type=jax.ShapeDtypeStruct((B, D_MODEL), jnp.float32),
        scratch_types=[
            pltpu.VMEM((n_chunks, _C), jnp.int32),
            row_buf, row_buf, row_buf, row_buf,
            pltpu.SemaphoreType.DMA, pltpu.SemaphoreType.DMA,
            pltpu.SemaphoreType.DMA, pltpu.SemaphoreType.DMA,
            pltpu.SemaphoreType.DMA, pltpu.SemaphoreType.DMA,
            pltpu.SemaphoreType.DMA, pltpu.SemaphoreType.DMA,
        ],
    )
    def k(x_hbm, lut_hbm, out_hbm, idx_v, r0, r1, r2, r3,
          g0, g1, g2, g3, s0, s1, s2, s3):
        rows = (r0, r1, r2, r3)
        gsem = (g0, g1, g2, g3)
        ssem = (s0, s1, s2, s3)
        wid = lax.axis_index("s") * nc + lax.axis_index("c")
        base = wid * b_per_w
        pltpu.sync_copy(x_hbm.at[wid], idx_v)

        def scale(buf):
            for r in range(_C):
                @plsc.parallel_loop(0, D_MODEL, step=_LANES, unroll=8)
                def _(i):
                    buf[r, pl.ds(i, _LANES)] = buf[r, pl.ds(i, _LANES)] * _SCALE

        def issue_gather(h, b):
            pltpu.async_copy(lut_hbm.at[idx_v.at[h]], rows[b], gsem[b])

        def step(g, tt, first_block, last_tt_cap):
            b = tt % _NBUF
            pltpu.make_async_copy(
                lut_hbm.at[pl.ds(0, _C)], rows[b], gsem[b]
            ).wait()
            scale(rows[b])
            pltpu.async_copy(
                rows[b], out_hbm.at[pl.ds(base + g * _C, _C)], ssem[b]
            )
            bh = (tt + _NBUF - 1) % _NBUF
            if last_tt_cap is None or tt < last_tt_cap:
                if not (first_block and tt == 0):
                    pltpu.make_async_copy(
                        rows[bh], out_hbm.at[pl.ds(0, _C)], ssem[bh]
                    ).wait()
                issue_gather(g + _NBUF - 1, bh)

        for h in range(_NBUF - 1):
            issue_gather(h, h % _NBUF)

        for tt in range(_INNER):
            step(tt, tt, True, None)

        def outer_body(o, carry):
            for tt in range(_INNER):
                step(o * _INNER + tt, tt, False, None)
            return carry

        lax.fori_loop(1, outer - 1, outer_body, 0)

        cap = _INNER - (_NBUF - 1)
        for tt in range(_INNER):
            step((outer - 1) * _INNER + tt, tt, False, cap)

        for b in range(_NBUF):
            pltpu.make_async_copy(
                rows[b], out_hbm.at[pl.ds(0, _C)], ssem[b]
            ).wait()

    return k


def kernel(x, lut):
    b0, b1 = x.shape
    info = plsc.get_sparse_core_info()
    nw = info.num_cores * info.num_subcores
    B = b0 * b1
    idx3 = x.reshape(nw, (B // nw) // _C, _C).astype(jnp.int32)
    out = _make_gather(B, lut.shape[0])(idx3, lut)
    return out.reshape(b0, b1, D_MODEL)
